# baseline (speedup 1.0000x reference)
import functools

import jax
import jax.numpy as jnp
from jax import lax
from jax.experimental import pallas as pl
from jax.experimental.pallas import tpu as pltpu
from jax.experimental.pallas import tpu_sc as plsc

NUM_CAT = 3
CAT_VOCAB = 100000
NUM_HASH = 3
HASH_BINS = 100000
NUM_CONT_EMB = 2
CONT_BINS = 1000
NUM_CONT_NORM = 2
EMB = 32
BATCH = 16384
OUT_COLS = NUM_CAT * EMB + NUM_HASH * EMB + NUM_CONT_EMB * EMB + NUM_CONT_NORM

_info = plsc.get_sparse_core_info()
NC = _info.num_cores
NS = _info.num_subcores
L = _info.num_lanes
NW = NC * NS
CHUNK = BATCH // NW
NSLOT = 4

BSLOT = 1024
PLEN = NUM_CONT_EMB * BSLOT + 64
MEAN_OFF = NUM_CONT_EMB * BSLOT
STD_OFF = MEAN_OFF + NUM_CONT_NORM


def _deinterleave(stag, ci, num_feat, lane):
    for f in range(num_feat):

        def body(j, _, f=f):
            rows = j * L + lane
            v = plsc.load_gather(stag, [rows * num_feat + f])
            ci[pl.ds(f * CHUNK + j * L, L)] = v
            return 0

        lax.fori_loop(0, CHUNK // L, body, 0)


def _tower_body(cat_idx_f, hash_idx_f, cont_vals_f, norm_vals_f,
                cat_tab, hash_tab, cont_tab, bparams,
                out,
                stag_i, stag_f, ci_cat, ci_hash, ci_cont, bp_v,
                rb0, rb1, rb2, rb3, normb, gsems, osems):
    rb = [rb0, rb1, rb2, rb3]
    wid = lax.axis_index("s") * NC + lax.axis_index("c")
    base = wid * CHUNK
    lane = lax.iota(jnp.int32, L)

    pltpu.sync_copy(cat_idx_f.at[pl.ds(base * NUM_CAT, CHUNK * NUM_CAT)],
                    stag_i)
    _deinterleave(stag_i, ci_cat, NUM_CAT, lane)
    g = [None] * 8
    o = [None] * 9
    for f in range(NUM_CAT):
        g[f] = pltpu.async_copy(
            cat_tab.at[f].at[ci_cat.at[pl.ds(f * CHUNK, CHUNK)]], rb[f],
            gsems.at[f])

    pltpu.sync_copy(hash_idx_f.at[pl.ds(base * NUM_HASH, CHUNK * NUM_HASH)],
                    stag_i)
    _deinterleave(stag_i, ci_hash, NUM_HASH, lane)
    g[3] = pltpu.async_copy(
        hash_tab.at[0].at[ci_hash.at[pl.ds(0, CHUNK)]], rb[3], gsems.at[3])

    pltpu.sync_copy(bparams, bp_v)
    pltpu.sync_copy(cont_vals_f.at[pl.ds(base * NUM_CONT_EMB,
                                         CHUNK * NUM_CONT_EMB)], stag_f)
    for f in range(NUM_CONT_EMB):
        fb = f * BSLOT
        b0 = plsc.load_gather(bp_v, [jnp.full((L,), fb + 1, jnp.int32)])
        blast = plsc.load_gather(
            bp_v, [jnp.full((L,), fb + CONT_BINS, jnp.int32)])
        ist = (CONT_BINS - 1.0) / (blast - b0)

        def body(j, _, f=f, fb=fb, b0=b0, ist=ist):
            rows = j * L + lane
            x = plsc.load_gather(stag_f, [rows * NUM_CONT_EMB + f])
            est = (x - b0) * ist
            est = jnp.minimum(jnp.maximum(est, -2.0), float(CONT_BINS + 1))
            c = est.astype(jnp.int32) + 1
            c = jnp.minimum(jnp.maximum(c, 0), CONT_BINS)
            for _ in range(2):
                lo = plsc.load_gather(bp_v, [fb + c])
                hi = plsc.load_gather(bp_v, [fb + c + 1])
                c = (c - (x < lo).astype(jnp.int32)
                     + (x >= hi).astype(jnp.int32))
            ci_cont[pl.ds(f * CHUNK + j * L, L)] = c
            return 0

        lax.fori_loop(0, CHUNK // L, body, 0)

    tail = [(hash_tab, ci_hash, 1), (hash_tab, ci_hash, 2),
            (cont_tab, ci_cont, 0), (cont_tab, ci_cont, 1)]
    for k in range(8):
        slot = k % NSLOT
        g[k].wait()
        o[k] = pltpu.async_copy(
            rb[slot],
            out.at[pl.ds(base, CHUNK), pl.ds(k * EMB, EMB)],
            osems.at[slot])
        if k + NSLOT < 8:
            o[k].wait()
            tab, cidx, f = tail[k]
            g[k + NSLOT] = pltpu.async_copy(
                tab.at[f].at[cidx.at[pl.ds(f * CHUNK, CHUNK)]],
                rb[slot], gsems.at[slot])
        if k == 3:
            pltpu.sync_copy(norm_vals_f.at[pl.ds(base * NUM_CONT_NORM,
                                                 CHUNK * NUM_CONT_NORM)],
                            stag_f)
            for f in range(NUM_CONT_NORM):
                f_vec = jnp.full((L,), f, jnp.int32)
                mv = plsc.load_gather(
                    bp_v, [jnp.full((L,), MEAN_OFF + f, jnp.int32)])
                sv = plsc.load_gather(
                    bp_v, [jnp.full((L,), STD_OFF + f, jnp.int32)])

                def nbody(j, _, f=f, f_vec=f_vec, mv=mv, sv=sv):
                    rows = j * L + lane
                    x = plsc.load_gather(stag_f,
                                         [rows * NUM_CONT_NORM + f])
                    plsc.store_scatter(normb, [rows, f_vec], (x - mv) / sv)
                    return 0

                lax.fori_loop(0, CHUNK // L, nbody, 0)
            o[8] = pltpu.async_copy(
                normb,
                out.at[pl.ds(base, CHUNK),
                       pl.ds((NUM_CAT + NUM_HASH + NUM_CONT_EMB) * EMB,
                             NUM_CONT_NORM)],
                osems.at[NSLOT])
    for k in range(4, 9):
        o[k].wait()


_tower = functools.partial(
    pl.kernel,
    mesh=plsc.VectorSubcoreMesh(core_axis_name="c", subcore_axis_name="s"),
    out_type=jax.ShapeDtypeStruct((BATCH, OUT_COLS), jnp.float32),
    scratch_types=[
        pltpu.VMEM((CHUNK * NUM_CAT,), jnp.int32),
        pltpu.VMEM((CHUNK * NUM_CONT_EMB,), jnp.float32),
        pltpu.VMEM((NUM_CAT * CHUNK,), jnp.int32),
        pltpu.VMEM((NUM_HASH * CHUNK,), jnp.int32),
        pltpu.VMEM((NUM_CONT_EMB * CHUNK,), jnp.int32),
        pltpu.VMEM((PLEN,), jnp.float32),
        pltpu.VMEM((CHUNK, EMB), jnp.float32),
        pltpu.VMEM((CHUNK, EMB), jnp.float32),
        pltpu.VMEM((CHUNK, EMB), jnp.float32),
        pltpu.VMEM((CHUNK, EMB), jnp.float32),
        pltpu.VMEM((CHUNK, NUM_CONT_NORM), jnp.float32),
        pltpu.SemaphoreType.DMA((NSLOT,)),
        pltpu.SemaphoreType.DMA((NSLOT + 1,)),
    ],
    compiler_params=pltpu.CompilerParams(
        use_tc_tiling_on_sc=False, needs_layout_passes=False),
)(_tower_body)


def kernel(cat_idx, hash_idx, cont_embed_vals, cont_norm_vals, cat_tables,
           hash_tables, cont_tables, cont_boundaries, norm_mean, norm_std):
    neg = jnp.full((1,), -jnp.inf, jnp.float32)
    pos = jnp.full((BSLOT - CONT_BINS - 1,), jnp.inf, jnp.float32)
    slots = [jnp.concatenate([neg, cont_boundaries[f], pos])
             for f in range(NUM_CONT_EMB)]
    tailpad = jnp.zeros((64 - 2 * NUM_CONT_NORM,), jnp.float32)
    bparams = jnp.concatenate(
        slots + [norm_mean.astype(jnp.float32),
                 norm_std.astype(jnp.float32), tailpad])
    return _tower(cat_idx.reshape(-1), hash_idx.reshape(-1),
                  cont_embed_vals.reshape(-1), cont_norm_vals.reshape(-1),
                  cat_tables, hash_tables, cont_tables, bparams)
